# Initial kernel scaffold; baseline (speedup 1.0000x reference)
import jax
import jax.numpy as jnp
from jax.experimental import pallas as pl


def kernel(x, edge_index, edge_type, rel_embed, w_in, w_out, w_loop, w_rel, loop_rel, bias, bn_gamma, bn_beta):
    raise NotImplementedError("write your pallas kernel here")


# speedup vs baseline: 11.3809x; 11.3809x over previous
import functools

import jax
import jax.numpy as jnp
from jax import lax
from jax.experimental import pallas as pl
from jax.experimental.pallas import tpu as pltpu
from jax.experimental.pallas import tpu_sc as plsc

NC = 2
NS = 16
LANES = 16


def _sc_mesh():
    return plsc.VectorSubcoreMesh(core_axis_name="c", subcore_axis_name="s",
                                  num_cores=NC, num_subcores=NS)


_SC_PARAMS = pltpu.CompilerParams(use_tc_tiling_on_sc=False)


def _build_deg_kernel(n_pad, nchunk, ch):
    nps = n_pad // NS
    sr = 128
    nrep = nps // sr
    f32 = jnp.float32

    @functools.partial(
        pl.kernel,
        out_type=jax.ShapeDtypeStruct((NC, n_pad, LANES), f32),
        mesh=_sc_mesh(),
        compiler_params=_SC_PARAMS,
        scratch_types=[
            pltpu.VMEM((nchunk, ch), jnp.int32),
            pltpu.VMEM((ch, LANES), f32),
            pltpu.VMEM((sr, LANES), f32),
            pltpu.VMEM_SHARED((n_pad, LANES), f32),
        ],
    )
    def deg_kernel(rows_hbm, deg_hbm, rowbuf, onesbuf, stage, acc):
        c = lax.axis_index("c")
        s = lax.axis_index("s")

        ones_v = jnp.ones((LANES,), f32)
        zero_v = jnp.zeros((LANES,), f32)

        def init_ones(i, carry):
            onesbuf[i, :] = ones_v
            stage[i, :] = zero_v
            return carry
        lax.fori_loop(0, max(ch, sr), init_ones, 0)

        for r in range(nrep):
            pltpu.sync_copy(stage, acc.at[pl.ds(s * nps + r * sr, sr)])
        plsc.subcore_barrier()

        pltpu.sync_copy(rows_hbm.at[c, s], rowbuf)

        def chunk(j, carry):
            pltpu.sync_copy(onesbuf, acc.at[rowbuf.at[j]], add=True)
            return carry
        lax.fori_loop(0, nchunk, chunk, 0)

        plsc.subcore_barrier()
        for r in range(nrep):
            pltpu.sync_copy(acc.at[pl.ds(s * nps + r * sr, sr)], stage)
            pltpu.sync_copy(stage, deg_hbm.at[c, pl.ds(s * nps + r * sr, sr)])

    return deg_kernel


def _build_agg_kernel(n_pad, d, nchunk, ch):
    nps = n_pad // NS
    sr = 16
    nrep = nps // sr
    f32 = jnp.float32
    dh = d // 2
    kgroups = dh // LANES

    @functools.partial(
        pl.kernel,
        out_type=jax.ShapeDtypeStruct((NC, n_pad, d), f32),
        mesh=_sc_mesh(),
        compiler_params=_SC_PARAMS,
        scratch_types=[
            pltpu.VMEM((nchunk, ch), jnp.int32),
            pltpu.VMEM((ch,), jnp.int32),
            pltpu.VMEM((ch,), jnp.int32),
            pltpu.VMEM((ch, d), f32),
            pltpu.VMEM((ch, d), f32),
            pltpu.VMEM((sr, d), f32),
            pltpu.VMEM_SHARED((n_pad, d), f32),
            pltpu.SemaphoreType.DMA,
            pltpu.SemaphoreType.DMA,
        ],
    )
    def agg_kernel(rows_hbm, cols_hbm, types_hbm, xs_hbm, rel_hbm, agg_hbm,
                   rowtab, colbuf, typbuf, xrows, rrows, stage, acc,
                   semx, semr):
        c = lax.axis_index("c")
        s = lax.axis_index("s")

        zero_v = jnp.zeros((LANES,), f32)

        def init_zero(i, carry):
            for k in range(d // LANES):
                stage[i, pl.ds(k * LANES, LANES)] = zero_v
            return carry
        lax.fori_loop(0, sr, init_zero, 0)

        for r in range(nrep):
            pltpu.sync_copy(stage, acc.at[pl.ds(s * nps + r * sr, sr)])
        plsc.subcore_barrier()

        pltpu.sync_copy(rows_hbm.at[c, s], rowtab)

        def chunk(j, carry):
            pltpu.sync_copy(cols_hbm.at[c, s, j], colbuf)
            pltpu.sync_copy(types_hbm.at[c, s, j], typbuf)
            cx = pltpu.async_copy(xs_hbm.at[colbuf], xrows, semx)
            cr = pltpu.async_copy(rel_hbm.at[typbuf], rrows, semr)
            cx.wait()
            cr.wait()

            def rot(e, carry2):
                for k in range(kgroups):
                    lo = pl.ds(k * LANES, LANES)
                    hi = pl.ds(dh + k * LANES, LANES)
                    hre = xrows[e, lo]
                    him = xrows[e, hi]
                    rre = rrows[e, lo]
                    rim = rrows[e, hi]
                    xrows[e, lo] = hre * rre - him * rim
                    xrows[e, hi] = hre * rim + him * rre
                return carry2
            lax.fori_loop(0, ch, rot, 0)

            pltpu.sync_copy(xrows, acc.at[rowtab.at[j]], add=True)
            return carry
        lax.fori_loop(0, nchunk, chunk, 0)

        plsc.subcore_barrier()
        for r in range(nrep):
            pltpu.sync_copy(acc.at[pl.ds(s * nps + r * sr, sr)], stage)
            pltpu.sync_copy(stage, agg_hbm.at[c, pl.ds(s * nps + r * sr, sr)])

    return agg_kernel


def _scale_body(x_ref, deg_ref, xs_ref):
    x = x_ref[...]
    for c in range(NC):
        deg = deg_ref[c][:, 0:1]
        dinv = jnp.where(deg > 0, lax.rsqrt(deg), 0.0)
        xs_ref[c] = x * dinv


def _dense_body(agg_ref, deg_ref, x_ref, rel_ref, win_ref, wout_ref,
                wloop_ref, wrel_ref, lrel_ref, bias_ref, g_ref, b_ref,
                ent_ref, rout_ref):
    f32 = jnp.float32
    n = deg_ref.shape[1]
    deg_in = deg_ref[0][:, 0:1]
    deg_out = deg_ref[1][:, 0:1]
    dinv_in = jnp.where(deg_in > 0, lax.rsqrt(deg_in), 0.0)
    dinv_out = jnp.where(deg_out > 0, lax.rsqrt(deg_out), 0.0)
    a_in = agg_ref[0][:n] * dinv_in
    a_out = agg_ref[1][:n] * dinv_out

    x = x_ref[...]
    d = x.shape[1]
    dh = d // 2
    lr = lrel_ref[0]
    hre, him = x[:, :dh], x[:, dh:]
    rre, rim = lr[:dh], lr[dh:]
    loop_m = jnp.concatenate([hre * rre - him * rim, hre * rim + him * rre],
                             axis=1)

    t = (jnp.dot(a_in, win_ref[...], preferred_element_type=f32)
         + jnp.dot(a_out, wout_ref[...], preferred_element_type=f32)
         + jnp.dot(loop_m, wloop_ref[...], preferred_element_type=f32))
    t = t * (1.0 / 3.0) + bias_ref[0]

    mean = jnp.mean(t, axis=0, keepdims=True)
    var = jnp.mean((t - mean) ** 2, axis=0, keepdims=True)
    ent_ref[...] = jnp.tanh((t - mean) * lax.rsqrt(var + 1e-5) * g_ref[0]
                            + b_ref[0])

    n_rel = rout_ref.shape[0]
    rout_ref[...] = jnp.dot(rel_ref[...], wrel_ref[...],
                            preferred_element_type=f32)[:n_rel]


def kernel(x, edge_index, edge_type, rel_embed, w_in, w_out, w_loop, w_rel,
           loop_rel, bias, bn_gamma, bn_beta):
    n_ent, d = x.shape
    e = edge_index.shape[1]
    eh = e // 2
    per_sub = eh // NS
    assert eh % NS == 0 and n_ent % NS == 0

    n_pad = -(-n_ent // (NS * 128)) * (NS * 128)
    ch = 128
    nchunk = -(-per_sub // ch)
    per_sub_pad = nchunk * ch
    padw = per_sub_pad - per_sub
    dummy = n_ent

    rows3 = edge_index[0].reshape(NC, NS, per_sub)
    cols3 = edge_index[1].reshape(NC, NS, per_sub)
    types3 = edge_type.reshape(NC, NS, per_sub)
    pw = ((0, 0), (0, 0), (0, padw))
    rows4 = jnp.pad(rows3, pw, constant_values=dummy).reshape(
        NC, NS, nchunk, ch)
    cols4 = jnp.pad(cols3, pw).reshape(NC, NS, nchunk, ch)
    types4 = jnp.pad(types3, pw).reshape(NC, NS, nchunk, ch)
    cols4 = cols4 + (jnp.arange(NC, dtype=jnp.int32) * n_pad)[
        :, None, None, None]
    rel_full = jnp.concatenate([rel_embed, loop_rel], axis=0)

    deg2 = _build_deg_kernel(n_pad, nchunk, ch)(rows4)[:, :n_ent]

    xs2 = pl.pallas_call(
        _scale_body,
        out_shape=jax.ShapeDtypeStruct((NC, n_ent, d), jnp.float32),
    )(x, deg2)

    xs_pad = jnp.pad(xs2, ((0, 0), (0, n_pad - n_ent), (0, 0)))
    agg2 = _build_agg_kernel(n_pad, d, nchunk, ch)(
        rows4, cols4, types4, xs_pad.reshape(NC * n_pad, d), rel_full)

    ent, rel_out = pl.pallas_call(
        _dense_body,
        out_shape=(
            jax.ShapeDtypeStruct((n_ent, d), jnp.float32),
            jax.ShapeDtypeStruct((rel_embed.shape[0], d), jnp.float32),
        ),
    )(agg2, deg2, x, rel_full, w_in, w_out, w_loop, w_rel, loop_rel,
      bias.reshape(1, d), bn_gamma.reshape(1, d), bn_beta.reshape(1, d))

    return ent, rel_out
